# Initial kernel scaffold; baseline (speedup 1.0000x reference)
import jax
import jax.numpy as jnp
from jax.experimental import pallas as pl


def kernel(x):
    raise NotImplementedError("write your pallas kernel here")


# speedup vs baseline: 22.4414x; 22.4414x over previous
import jax
import jax.numpy as jnp
from jax import lax
from jax.experimental import pallas as pl
from jax.experimental.pallas import tpu as pltpu

_ROWS = 8192
_COLS = 1024
_N = _ROWS * _COLS
_K = max(1, int(_N * 0.01))
_NBLK = 32
_RB = _ROWS // _NBLK


def _tc_body(x_ref, o_ref, t_ref):
    i = pl.program_id(0)

    @pl.when(i == 0)
    def _search():
        def step(j, t):
            cand = t | (jnp.int32(1) << (jnp.int32(30) - j))

            def chunk(c, acc):
                xc = x_ref[pl.ds(c * _RB, _RB), :]
                uc = lax.bitcast_convert_type(xc, jnp.int32) & jnp.int32(0x7FFFFFFF)
                return acc + jnp.sum((uc >= cand).astype(jnp.int32))

            cnt = lax.fori_loop(0, _NBLK, chunk, jnp.int32(0))
            return jnp.where(cnt >= _K, cand, t)

        t_ref[0] = lax.fori_loop(0, 31, step, jnp.int32(0))

    t = t_ref[0]
    xb = x_ref[pl.ds(i * _RB, _RB), :]
    ub = lax.bitcast_convert_type(xb, jnp.int32) & jnp.int32(0x7FFFFFFF)
    o_ref[...] = jnp.where(ub >= t, xb, jnp.float32(0.0))


def kernel(x):
    flat = x.reshape(_ROWS, _COLS)
    out = pl.pallas_call(
        _tc_body,
        grid=(_NBLK,),
        in_specs=[pl.BlockSpec((_ROWS, _COLS), lambda i: (0, 0))],
        out_specs=pl.BlockSpec((_RB, _COLS), lambda i: (i, 0)),
        out_shape=jax.ShapeDtypeStruct((_ROWS, _COLS), jnp.float32),
        scratch_shapes=[pltpu.SMEM((1,), jnp.int32)],
    )(flat)
    return out.reshape(x.shape)
